# baseline (speedup 1.0000x reference)
import functools

import jax
import jax.numpy as jnp
from jax import lax
from jax.experimental import pallas as pl
from jax.experimental.pallas import tpu as pltpu
from jax.experimental.pallas import tpu_sc as plsc

B = 2
C = 4
NPB = 64 * 128 * 128
N = B * NPB
K = max(1, int(0.3 * N))

SUB = NPB // 8
WCOL = 65536


def _loss_body(lg_ref, tg_ref, out_ref):
    x = lg_ref[0]
    t = tg_ref[0]
    m = jnp.max(x, axis=0)
    s = jnp.sum(jnp.exp(x - m[None]), axis=0)
    lse = m + jnp.log(s)
    cidx = lax.broadcasted_iota(jnp.int32, (C, 8, WCOL), 0)
    sel = jnp.sum(jnp.where(cidx == t[None], x, 0.0), axis=0)
    out_ref[0] = jnp.maximum(lse - sel, 0.0)


_loss_call = pl.pallas_call(
    _loss_body,
    grid=(B, SUB // WCOL),
    in_specs=[
        pl.BlockSpec((1, C, 8, WCOL), lambda b, j: (b, 0, 0, j)),
        pl.BlockSpec((1, 8, WCOL), lambda b, j: (b, 0, j)),
    ],
    out_specs=pl.BlockSpec((1, 8, WCOL), lambda b, j: (b, 0, j)),
    out_shape=jax.ShapeDtypeStruct((B, 8, SUB), jnp.float32),
)

NC = 2
NS = 16
NW = NC * NS
L = 16
PER_W = N // NW
CH = 16384
NCH = PER_W // CH
NB = 4096
SHIFT = 19
UNROLL = 8


def _hist_body(loss_hbm, cnt_out, stage0, stage1, ctbl, row, sem0, sem1):
    stages = (stage0, stage1)
    sems = (sem0, sem1)
    wid = lax.axis_index("s") * NC + lax.axis_index("c")
    base = wid * PER_W

    zv = jnp.zeros((L,), jnp.float32)

    @plsc.parallel_loop(0, NB, unroll=UNROLL)
    def _(i):
        ctbl[pl.ds(i * L, L)] = zv

    lane_off = jnp.arange(L, dtype=jnp.int32) * NB
    ones = jnp.full((L,), 1.0, jnp.float32)

    def dma(g):
        return pltpu.make_async_copy(
            loss_hbm.at[pl.ds(base + g * CH, CH)],
            stages[g % 2], sems[g % 2])

    def process(sref):
        @plsc.parallel_loop(0, CH // L, unroll=UNROLL)
        def _(i):
            v = sref[pl.ds(i * L, L)]
            bits = lax.bitcast_convert_type(v, jnp.int32)
            bucket = lax.shift_right_logical(bits, SHIFT)
            plsc.addupdate_scatter(
                ctbl, [bucket + lane_off], ones,
                mask=jnp.full((L,), True))

    dma(0).start()
    for g in range(NCH):
        if g + 1 < NCH:
            dma(g + 1).start()
        dma(g).wait()
        process(stages[g % 2])

    @plsc.parallel_loop(0, NB // L, unroll=4)
    def _(j):
        acc = ctbl[pl.ds(j * L, L)]
        for l in range(1, L):
            acc = acc + ctbl[pl.ds(l * NB + j * L, L)]
        row[pl.ds(j * L, L)] = acc

    pltpu.sync_copy(row, cnt_out.at[wid])


@functools.lru_cache(maxsize=1)
def _get_hist_kernel():
    mesh = plsc.VectorSubcoreMesh(core_axis_name="c", subcore_axis_name="s")
    return functools.partial(
        pl.kernel, mesh=mesh,
        out_type=jax.ShapeDtypeStruct((NW, NB), jnp.float32),
        scratch_types=[
            pltpu.VMEM((CH,), jnp.float32),
            pltpu.VMEM((CH,), jnp.float32),
            pltpu.VMEM((NB * L,), jnp.float32),
            pltpu.VMEM((NB,), jnp.float32),
            pltpu.SemaphoreType.DMA,
            pltpu.SemaphoreType.DMA,
        ],
        compiler_params=pltpu.CompilerParams(needs_layout_passes=False),
    )(_hist_body)


NR = NB // 128
FBLK = 524288
FSTEPS = N // FBLK


def _find_bin(cnt3, kneed):
    cnt = jnp.sum(cnt3, axis=0)
    tri = (lax.broadcasted_iota(jnp.int32, (128, 128), 0)
           >= lax.broadcasted_iota(jnp.int32, (128, 128), 1)
           ).astype(jnp.float32)
    strict = (lax.broadcasted_iota(jnp.int32, (NR, NR), 1)
              > lax.broadcasted_iota(jnp.int32, (NR, NR), 0)
              ).astype(jnp.float32)
    srow = jnp.dot(cnt, tri, preferred_element_type=jnp.float32)
    rt = jnp.sum(cnt, axis=1).reshape(1, NR)
    s_cnt = srow + jnp.sum(strict * rt, axis=1, keepdims=True)
    fidx = (lax.broadcasted_iota(jnp.int32, (NR, 128), 0) * 128
            + lax.broadcasted_iota(jnp.int32, (NR, 128), 1)
            ).astype(jnp.float32)
    return jnp.max(jnp.where(s_cnt >= kneed, fidx, -1.0))


def _final_body(cnt_ref, loss_ref, out_ref, edges, acc):
    g = pl.program_id(0)

    @pl.when(g == 0)
    def _():
        b = _find_bin(cnt_ref[...], float(K)).astype(jnp.int32)
        edges[0] = b << SHIFT
        edges[1] = (b << SHIFT) | (1 << (SHIFT - 1))
        edges[2] = (b + 1) << SHIFT
        for i in range(6):
            acc[i] = 0.0

    x = loss_ref[0]
    xb = lax.bitcast_convert_type(x, jnp.int32)
    for j in range(3):
        m = xb >= edges[j]
        acc[2 * j] = acc[2 * j] + jnp.sum(jnp.where(m, x, 0.0))
        acc[2 * j + 1] = acc[2 * j + 1] + jnp.sum(m.astype(jnp.float32))

    @pl.when(g == FSTEPS - 1)
    def _():
        upper = acc[3] >= float(K)
        s_gt = jnp.where(upper, acc[4], acc[2])
        c_gt = jnp.where(upper, acc[5], acc[3])
        in_sum = jnp.where(upper, acc[2] - acc[4], acc[0] - acc[2])
        in_cnt = jnp.where(upper, acc[3] - acc[5], acc[1] - acc[3])
        k_rem = float(K) - c_gt
        out_ref[0, 0] = (s_gt + k_rem * (in_sum / in_cnt)) / float(K)


_final = pl.pallas_call(
    _final_body,
    grid=(FSTEPS,),
    in_specs=[
        pl.BlockSpec((NW, NR, 128), lambda g: (0, 0, 0)),
        pl.BlockSpec((1, 8, FBLK // 8), lambda g: (g, 0, 0)),
    ],
    out_specs=pl.BlockSpec(
        (1, 1), lambda g: (0, 0), memory_space=pltpu.SMEM),
    out_shape=jax.ShapeDtypeStruct((1, 1), jnp.float32),
    scratch_shapes=[pltpu.SMEM((3,), jnp.int32),
                    pltpu.SMEM((6,), jnp.float32)],
)


def kernel(logits, target):
    lg = logits.reshape(B, C, 8, SUB)
    tg = target.astype(jnp.int32).reshape(B, 8, SUB)
    losses = _loss_call(lg, tg).reshape(N)
    cnt = _get_hist_kernel()(losses)
    res = _final(cnt.reshape(NW, NR, 128),
                 losses.reshape(FSTEPS, 8, FBLK // 8))
    return res[0, 0]


# speedup vs baseline: 1.0131x; 1.0131x over previous
import functools

import jax
import jax.numpy as jnp
from jax import lax
from jax.experimental import pallas as pl
from jax.experimental.pallas import tpu as pltpu
from jax.experimental.pallas import tpu_sc as plsc

B = 2
C = 4
NPB = 64 * 128 * 128
N = B * NPB
K = max(1, int(0.3 * N))

SUB = NPB // 8
WCOL = 65536


def _loss_body(lg_ref, tg_ref, out_ref):
    x = lg_ref[0]
    t = tg_ref[0]
    m = jnp.max(x, axis=0)
    s = jnp.sum(jnp.exp(x - m[None]), axis=0)
    lse = m + jnp.log(s)
    cidx = lax.broadcasted_iota(jnp.int32, (C, 8, WCOL), 0)
    sel = jnp.sum(jnp.where(cidx == t[None], x, 0.0), axis=0)
    out_ref[0] = jnp.maximum(lse - sel, 0.0)


_loss_call = pl.pallas_call(
    _loss_body,
    grid=(B, SUB // WCOL),
    in_specs=[
        pl.BlockSpec((1, C, 8, WCOL), lambda b, j: (b, 0, 0, j)),
        pl.BlockSpec((1, 8, WCOL), lambda b, j: (b, 0, j)),
    ],
    out_specs=pl.BlockSpec((1, 8, WCOL), lambda b, j: (b, 0, j)),
    out_shape=jax.ShapeDtypeStruct((B, 8, SUB), jnp.float32),
)

NC = 2
NS = 16
NW = NC * NS
L = 16
PER_W = N // NW
CH = 16384
NCH = PER_W // CH
NB = 4096
SHIFT = 19
UNROLL = 8


def _hist_body(loss_hbm, cnt_out, stage0, stage1, ctbl, row, sem0, sem1):
    stages = (stage0, stage1)
    sems = (sem0, sem1)
    wid = lax.axis_index("s") * NC + lax.axis_index("c")
    base = wid * PER_W

    zv = jnp.zeros((L,), jnp.float32)

    @plsc.parallel_loop(0, NB, unroll=UNROLL)
    def _(i):
        ctbl[pl.ds(i * L, L)] = zv

    lane_off = jnp.arange(L, dtype=jnp.int32) * NB
    ones = jnp.full((L,), 1.0, jnp.float32)

    def dma(g):
        return pltpu.make_async_copy(
            loss_hbm.at[pl.ds(base + g * CH, CH)],
            stages[g % 2], sems[g % 2])

    def process(sref):
        @plsc.parallel_loop(0, CH // L, unroll=UNROLL)
        def _(i):
            v = sref[pl.ds(i * L, L)]
            bits = lax.bitcast_convert_type(v, jnp.int32)
            bucket = lax.shift_right_logical(bits, SHIFT)
            plsc.addupdate_scatter(
                ctbl, [bucket + lane_off], ones,
                mask=jnp.full((L,), True))

    dma(0).start()
    for g in range(NCH):
        if g + 1 < NCH:
            dma(g + 1).start()
        dma(g).wait()
        process(stages[g % 2])

    @plsc.parallel_loop(0, NB // L, unroll=4)
    def _(j):
        acc = ctbl[pl.ds(j * L, L)]
        for l in range(1, L):
            acc = acc + ctbl[pl.ds(l * NB + j * L, L)]
        row[pl.ds(j * L, L)] = acc

    pltpu.sync_copy(row, cnt_out.at[wid])


@functools.lru_cache(maxsize=1)
def _get_hist_kernel():
    mesh = plsc.VectorSubcoreMesh(core_axis_name="c", subcore_axis_name="s")
    return functools.partial(
        pl.kernel, mesh=mesh,
        out_type=jax.ShapeDtypeStruct((NW, NB), jnp.float32),
        scratch_types=[
            pltpu.VMEM((CH,), jnp.float32),
            pltpu.VMEM((CH,), jnp.float32),
            pltpu.VMEM((NB * L,), jnp.float32),
            pltpu.VMEM((NB,), jnp.float32),
            pltpu.SemaphoreType.DMA,
            pltpu.SemaphoreType.DMA,
        ],
        compiler_params=pltpu.CompilerParams(needs_layout_passes=False),
    )(_hist_body)


NR = NB // 128
FBLK = 262144
FSTEPS = N // FBLK


def _find_bin(cnt3, kneed):
    cnt = jnp.sum(cnt3, axis=0)
    tri = (lax.broadcasted_iota(jnp.int32, (128, 128), 0)
           >= lax.broadcasted_iota(jnp.int32, (128, 128), 1)
           ).astype(jnp.float32)
    strict = (lax.broadcasted_iota(jnp.int32, (NR, NR), 1)
              > lax.broadcasted_iota(jnp.int32, (NR, NR), 0)
              ).astype(jnp.float32)
    srow = jnp.dot(cnt, tri, preferred_element_type=jnp.float32)
    rt = jnp.sum(cnt, axis=1).reshape(1, NR)
    s_cnt = srow + jnp.sum(strict * rt, axis=1, keepdims=True)
    fidx = (lax.broadcasted_iota(jnp.int32, (NR, 128), 0) * 128
            + lax.broadcasted_iota(jnp.int32, (NR, 128), 1)
            ).astype(jnp.float32)
    return jnp.max(jnp.where(s_cnt >= kneed, fidx, -1.0))


def _final_body(cnt_ref, loss_ref, out_ref, edges, acc):
    g = pl.program_id(0)

    @pl.when(g == 0)
    def _():
        b = _find_bin(cnt_ref[...], float(K)).astype(jnp.int32)
        edges[0] = b << SHIFT
        edges[1] = (b + 1) << SHIFT
        for i in range(4):
            acc[i] = 0.0

    x = loss_ref[0]
    xb = lax.bitcast_convert_type(x, jnp.int32)
    for j in range(2):
        m = xb >= edges[j]
        acc[2 * j] = acc[2 * j] + jnp.sum(jnp.where(m, x, 0.0))
        acc[2 * j + 1] = acc[2 * j + 1] + jnp.sum(m.astype(jnp.float32))

    @pl.when(g == FSTEPS - 1)
    def _():
        s_gt, c_gt = acc[2], acc[3]
        in_sum = acc[0] - acc[2]
        in_cnt = acc[1] - acc[3]
        k_rem = float(K) - c_gt
        out_ref[0, 0] = (s_gt + k_rem * (in_sum / in_cnt)) / float(K)


_final = pl.pallas_call(
    _final_body,
    grid=(FSTEPS,),
    in_specs=[
        pl.BlockSpec((NW, NR, 128), lambda g: (0, 0, 0)),
        pl.BlockSpec((1, 8, FBLK // 8), lambda g: (g, 0, 0)),
    ],
    out_specs=pl.BlockSpec(
        (1, 1), lambda g: (0, 0), memory_space=pltpu.SMEM),
    out_shape=jax.ShapeDtypeStruct((1, 1), jnp.float32),
    scratch_shapes=[pltpu.SMEM((2,), jnp.int32),
                    pltpu.SMEM((4,), jnp.float32)],
)


def kernel(logits, target):
    lg = logits.reshape(B, C, 8, SUB)
    tg = target.astype(jnp.int32).reshape(B, 8, SUB)
    losses = _loss_call(lg, tg).reshape(N)
    cnt = _get_hist_kernel()(losses)
    res = _final(cnt.reshape(NW, NR, 128),
                 losses.reshape(FSTEPS, 8, FBLK // 8))
    return res[0, 0]
